# baseline (speedup 1.0000x reference)
import functools

import jax
import jax.numpy as jnp
from jax.experimental import pallas as pl
from jax.experimental.pallas import tpu as pltpu

_TOKENS = 8192
_HIDDEN = 2048
_H4 = _HIDDEN // 4
_E = 16
_MAXK = 4
_BT = 512
_NBLK = _TOKENS // _BT
_WROWS = _H4 + _E


def _tail(j, lt, zt, logits_ref, idxt_ref, wtst_ref, acc_load, acc_ent):
    col = j * _BT
    k_hi = zt > 0.0

    iota_s = jax.lax.broadcasted_iota(jnp.int32, (_E, _BT), 0)
    work = lt
    tvs, tis = [], []
    for _ in range(_MAXK):
        m = jnp.max(work, axis=0, keepdims=True)
        am = jnp.min(jnp.where(work == m, iota_s, _E), axis=0, keepdims=True)
        tvs.append(m)
        tis.append(am)
        work = jnp.where(iota_s == am, -jnp.inf, work)
    tv = jnp.concatenate(tvs, axis=0)
    ti = jnp.concatenate(tis, axis=0)

    iota4 = jax.lax.broadcasted_iota(jnp.int32, (_MAXK, _BT), 0)
    mask = jnp.logical_or(iota4 == 0, k_hi)
    idxt_ref[pl.ds(col, _BT), :] = jnp.where(mask, ti, 0).T
    ex = jnp.where(mask, jnp.exp(tv - tvs[0]), 0.0)
    wtst_ref[pl.ds(col, _BT), :] = (ex / jnp.sum(ex, axis=0, keepdims=True)).T

    logits_ref[pl.ds(col, _BT), :] = lt.T

    pe = jnp.exp(lt - tvs[0])
    probs = pe / jnp.sum(pe, axis=0, keepdims=True)
    acc_load[...] += jnp.sum(probs, axis=1, keepdims=True)
    acc_ent[0, 0] += -jnp.sum(probs * jnp.log(probs + 1e-08))


def _router_body(x_ref, w1_ref, b1r_ref, w2_ref, b2_ref, wr_ref,
                 logits_ref, idxt_ref, wtst_ref, var_ref, ent_ref,
                 wcat, lt_buf, zt_buf, acc_load, acc_ent):
    i = pl.program_id(0)

    @pl.when(i == 0)
    def _():
        wcat[0:_H4, :] = w1_ref[...].astype(jnp.bfloat16)
        wcat[_H4:_WROWS, :] = wr_ref[...].astype(jnp.bfloat16)
        acc_load[...] = jnp.zeros_like(acc_load)
        acc_ent[0, 0] = 0.0

    x = x_ref[...].astype(jnp.bfloat16)
    yt = jax.lax.dot_general(wcat[...], x, (((1,), (1,)), ((), ())),
                             preferred_element_type=jnp.float32)
    b1c = b1r_ref[...].T
    ht = jnp.maximum(yt[0:_H4, :] + b1c, 0.0)
    z8 = jax.lax.dot_general(w2_ref[...].astype(jnp.bfloat16),
                             ht.astype(jnp.bfloat16),
                             (((1,), (0,)), ((), ())),
                             preferred_element_type=jnp.float32)
    lt_buf[i % 2] = yt[_H4:_WROWS, :]
    zt_buf[i % 2] = z8[0:1, :] + b2_ref[0]

    tail = functools.partial(_tail, logits_ref=logits_ref,
                             idxt_ref=idxt_ref, wtst_ref=wtst_ref,
                             acc_load=acc_load, acc_ent=acc_ent)

    @pl.when(i > 0)
    def _():
        tail(i - 1, lt_buf[(i - 1) % 2], zt_buf[(i - 1) % 2])

    @pl.when(i == _NBLK - 1)
    def _():
        tail(_NBLK - 1, lt_buf[(_NBLK - 1) % 2], zt_buf[(_NBLK - 1) % 2])
        el = acc_load[...] / _TOKENS
        mu = jnp.sum(el) / _E
        var = jnp.sum((el - mu) ** 2) / (_E - 1)
        var_ref[...] = jnp.full((1, 1), var, jnp.float32)
        ent_ref[...] = jnp.full((1, 1), acc_ent[0, 0] / _TOKENS, jnp.float32)


@jax.jit
def _router(hidden_states, Wr, W1, b1, W2, b2):
    x = hidden_states
    b1r = b1.reshape(1, _H4)

    out_shapes = (
        jax.ShapeDtypeStruct((_TOKENS, _E), jnp.float32),
        jax.ShapeDtypeStruct((_TOKENS, _MAXK), jnp.int32),
        jax.ShapeDtypeStruct((_TOKENS, _MAXK), jnp.float32),
        jax.ShapeDtypeStruct((1, 1), jnp.float32),
        jax.ShapeDtypeStruct((1, 1), jnp.float32),
    )
    in_specs = [
        pl.BlockSpec((_BT, _HIDDEN), lambda i: (i, 0)),
        pl.BlockSpec((_H4, _HIDDEN), lambda i: (0, 0)),
        pl.BlockSpec((1, _H4), lambda i: (0, 0)),
        pl.BlockSpec((1, _H4), lambda i: (0, 0)),
        pl.BlockSpec(memory_space=pltpu.SMEM),
        pl.BlockSpec((_E, _HIDDEN), lambda i: (0, 0)),
    ]
    out_specs = (
        pl.BlockSpec((_TOKENS, _E), lambda i: (0, 0)),
        pl.BlockSpec((_TOKENS, _MAXK), lambda i: (0, 0)),
        pl.BlockSpec((_TOKENS, _MAXK), lambda i: (0, 0)),
        pl.BlockSpec((1, 1), lambda i: (0, 0)),
        pl.BlockSpec((1, 1), lambda i: (0, 0)),
    )
    return pl.pallas_call(
        _router_body,
        grid=(_NBLK,),
        in_specs=in_specs,
        out_specs=out_specs,
        out_shape=out_shapes,
        scratch_shapes=[
            pltpu.VMEM((_WROWS, _HIDDEN), jnp.bfloat16),
            pltpu.VMEM((2, _E, _BT), jnp.float32),
            pltpu.VMEM((2, 1, _BT), jnp.float32),
            pltpu.VMEM((_E, 1), jnp.float32),
            pltpu.SMEM((1, 1), jnp.float32),
        ],
    )(x, W1, b1r, W2, b2, Wr)


def kernel(hidden_states, Wr, W1, b1, W2, b2):
    logits, idx, wts, var, ent = _router(hidden_states, Wr, W1, b1, W2, b2)
    return (logits, idx.astype(jnp.int64), wts, var[0, 0], ent[0, 0])


# speedup vs baseline: 1.0960x; 1.0960x over previous
import functools

import jax
import jax.numpy as jnp
from jax.experimental import pallas as pl
from jax.experimental.pallas import tpu as pltpu

_TOKENS = 8192
_HIDDEN = 2048
_H4 = _HIDDEN // 4
_E = 16
_MAXK = 4
_BT = 1024
_NBLK = _TOKENS // _BT
_WROWS = _H4 + _E


def _tail(j, lt, zt, logits_ref, idxt_ref, wtst_ref, acc_load, acc_ent):
    col = j * _BT
    k_hi = zt > 0.0

    iota_s = jax.lax.broadcasted_iota(jnp.int32, (_E, _BT), 0)
    work = lt
    tvs, tis = [], []
    for _ in range(_MAXK):
        m = jnp.max(work, axis=0, keepdims=True)
        am = jnp.min(jnp.where(work == m, iota_s, _E), axis=0, keepdims=True)
        tvs.append(m)
        tis.append(am)
        work = jnp.where(iota_s == am, -jnp.inf, work)
    tv = jnp.concatenate(tvs, axis=0)
    ti = jnp.concatenate(tis, axis=0)

    iota4 = jax.lax.broadcasted_iota(jnp.int32, (_MAXK, _BT), 0)
    mask = jnp.logical_or(iota4 == 0, k_hi)
    idxt_ref[pl.ds(col, _BT), :] = jnp.where(mask, ti, 0).T
    ex = jnp.where(mask, jnp.exp(tv - tvs[0]), 0.0)
    wtst_ref[pl.ds(col, _BT), :] = (ex / jnp.sum(ex, axis=0, keepdims=True)).T

    logits_ref[pl.ds(col, _BT), :] = lt.T

    pe = jnp.exp(lt - tvs[0])
    probs = pe / jnp.sum(pe, axis=0, keepdims=True)
    acc_load[...] += jnp.sum(probs, axis=1, keepdims=True)
    acc_ent[0, 0] += -jnp.sum(probs * jnp.log(probs + 1e-08))


def _router_body(x_ref, w1_ref, b1r_ref, w2_ref, b2_ref, wr_ref,
                 logits_ref, idxt_ref, wtst_ref, var_ref, ent_ref,
                 wcat, lt_buf, zt_buf, acc_load, acc_ent):
    i = pl.program_id(0)

    @pl.when(i == 0)
    def _():
        wcat[0:_H4, :] = w1_ref[...].astype(jnp.bfloat16)
        wcat[_H4:_WROWS, :] = wr_ref[...].astype(jnp.bfloat16)
        acc_load[...] = jnp.zeros_like(acc_load)
        acc_ent[0, 0] = 0.0

    x = x_ref[...].astype(jnp.bfloat16)
    yt = jax.lax.dot_general(wcat[...], x, (((1,), (1,)), ((), ())),
                             preferred_element_type=jnp.float32)
    b1c = b1r_ref[...].T
    ht = jnp.maximum(yt[0:_H4, :] + b1c, 0.0)
    z8 = jax.lax.dot_general(w2_ref[...].astype(jnp.bfloat16),
                             ht.astype(jnp.bfloat16),
                             (((1,), (0,)), ((), ())),
                             preferred_element_type=jnp.float32)
    lt_buf[i % 2] = yt[_H4:_WROWS, :]
    zt_buf[i % 2] = z8[0:1, :] + b2_ref[0]

    tail = functools.partial(_tail, logits_ref=logits_ref,
                             idxt_ref=idxt_ref, wtst_ref=wtst_ref,
                             acc_load=acc_load, acc_ent=acc_ent)

    @pl.when(i > 0)
    def _():
        tail(i - 1, lt_buf[(i - 1) % 2], zt_buf[(i - 1) % 2])

    @pl.when(i == _NBLK - 1)
    def _():
        tail(_NBLK - 1, lt_buf[(_NBLK - 1) % 2], zt_buf[(_NBLK - 1) % 2])
        el = acc_load[...] / _TOKENS
        mu = jnp.sum(el) / _E
        var = jnp.sum((el - mu) ** 2) / (_E - 1)
        var_ref[...] = jnp.full((1, 1), var, jnp.float32)
        ent_ref[...] = jnp.full((1, 1), acc_ent[0, 0] / _TOKENS, jnp.float32)


@jax.jit
def _router(hidden_states, Wr, W1, b1, W2, b2):
    x = hidden_states
    b1r = b1.reshape(1, _H4)

    out_shapes = (
        jax.ShapeDtypeStruct((_TOKENS, _E), jnp.float32),
        jax.ShapeDtypeStruct((_TOKENS, _MAXK), jnp.int32),
        jax.ShapeDtypeStruct((_TOKENS, _MAXK), jnp.float32),
        jax.ShapeDtypeStruct((1, 1), jnp.float32),
        jax.ShapeDtypeStruct((1, 1), jnp.float32),
    )
    in_specs = [
        pl.BlockSpec((_BT, _HIDDEN), lambda i: (i, 0)),
        pl.BlockSpec((_H4, _HIDDEN), lambda i: (0, 0)),
        pl.BlockSpec((1, _H4), lambda i: (0, 0)),
        pl.BlockSpec((1, _H4), lambda i: (0, 0)),
        pl.BlockSpec(memory_space=pltpu.SMEM),
        pl.BlockSpec((_E, _HIDDEN), lambda i: (0, 0)),
    ]
    out_specs = (
        pl.BlockSpec((_TOKENS, _E), lambda i: (0, 0)),
        pl.BlockSpec((_TOKENS, _MAXK), lambda i: (0, 0)),
        pl.BlockSpec((_TOKENS, _MAXK), lambda i: (0, 0)),
        pl.BlockSpec((1, 1), lambda i: (0, 0)),
        pl.BlockSpec((1, 1), lambda i: (0, 0)),
    )
    return pl.pallas_call(
        _router_body,
        grid=(_NBLK,),
        in_specs=in_specs,
        out_specs=out_specs,
        out_shape=out_shapes,
        scratch_shapes=[
            pltpu.VMEM((_WROWS, _HIDDEN), jnp.bfloat16),
            pltpu.VMEM((2, _E, _BT), jnp.float32),
            pltpu.VMEM((2, 1, _BT), jnp.float32),
            pltpu.VMEM((_E, 1), jnp.float32),
            pltpu.SMEM((1, 1), jnp.float32),
        ],
    )(x, W1, b1r, W2, b2, Wr)


def kernel(hidden_states, Wr, W1, b1, W2, b2):
    logits, idx, wts, var, ent = _router(hidden_states, Wr, W1, b1, W2, b2)
    return (logits, idx.astype(jnp.int64), wts, var[0, 0], ent[0, 0])


# speedup vs baseline: 1.0984x; 1.0021x over previous
import functools

import jax
import jax.numpy as jnp
from jax.experimental import pallas as pl
from jax.experimental.pallas import tpu as pltpu

_TOKENS = 8192
_HIDDEN = 2048
_H4 = _HIDDEN // 4
_E = 16
_MAXK = 4
_BT = 2048
_NBLK = _TOKENS // _BT
_WROWS = _H4 + _E


def _tail(j, lt, zt, logits_ref, idxt_ref, wtst_ref, acc_load, acc_ent):
    col = j * _BT
    k_hi = zt > 0.0

    iota_s = jax.lax.broadcasted_iota(jnp.int32, (_E, _BT), 0)
    work = lt
    tvs, tis = [], []
    for _ in range(_MAXK):
        m = jnp.max(work, axis=0, keepdims=True)
        am = jnp.min(jnp.where(work == m, iota_s, _E), axis=0, keepdims=True)
        tvs.append(m)
        tis.append(am)
        work = jnp.where(iota_s == am, -jnp.inf, work)
    tv = jnp.concatenate(tvs, axis=0)
    ti = jnp.concatenate(tis, axis=0)

    iota4 = jax.lax.broadcasted_iota(jnp.int32, (_MAXK, _BT), 0)
    mask = jnp.logical_or(iota4 == 0, k_hi)
    idxt_ref[pl.ds(col, _BT), :] = jnp.where(mask, ti, 0).T
    ex = jnp.where(mask, jnp.exp(tv - tvs[0]), 0.0)
    wtst_ref[pl.ds(col, _BT), :] = (ex / jnp.sum(ex, axis=0, keepdims=True)).T

    logits_ref[pl.ds(col, _BT), :] = lt.T

    pe = jnp.exp(lt - tvs[0])
    probs = pe / jnp.sum(pe, axis=0, keepdims=True)
    acc_load[...] += jnp.sum(probs, axis=1, keepdims=True)
    acc_ent[0, 0] += -jnp.sum(probs * jnp.log(probs + 1e-08))


def _router_body(x_ref, w1_ref, b1r_ref, w2_ref, b2_ref, wr_ref,
                 logits_ref, idxt_ref, wtst_ref, var_ref, ent_ref,
                 wcat, lt_buf, zt_buf, acc_load, acc_ent):
    i = pl.program_id(0)

    @pl.when(i == 0)
    def _():
        wcat[0:_H4, :] = w1_ref[...].astype(jnp.bfloat16)
        wcat[_H4:_WROWS, :] = wr_ref[...].astype(jnp.bfloat16)
        acc_load[...] = jnp.zeros_like(acc_load)
        acc_ent[0, 0] = 0.0

    x = x_ref[...].astype(jnp.bfloat16)
    yt = jax.lax.dot_general(wcat[...], x, (((1,), (1,)), ((), ())),
                             preferred_element_type=jnp.float32)
    b1c = b1r_ref[...].T
    ht = jnp.maximum(yt[0:_H4, :] + b1c, 0.0)
    z8 = jax.lax.dot_general(w2_ref[...].astype(jnp.bfloat16),
                             ht.astype(jnp.bfloat16),
                             (((1,), (0,)), ((), ())),
                             preferred_element_type=jnp.float32)
    lt_buf[i % 2] = yt[_H4:_WROWS, :]
    zt_buf[i % 2] = z8[0:1, :] + b2_ref[0]

    tail = functools.partial(_tail, logits_ref=logits_ref,
                             idxt_ref=idxt_ref, wtst_ref=wtst_ref,
                             acc_load=acc_load, acc_ent=acc_ent)

    @pl.when(i > 0)
    def _():
        tail(i - 1, lt_buf[(i - 1) % 2], zt_buf[(i - 1) % 2])

    @pl.when(i == _NBLK - 1)
    def _():
        tail(_NBLK - 1, lt_buf[(_NBLK - 1) % 2], zt_buf[(_NBLK - 1) % 2])
        el = acc_load[...] / _TOKENS
        mu = jnp.sum(el) / _E
        var = jnp.sum((el - mu) ** 2) / (_E - 1)
        var_ref[...] = jnp.full((1, 1), var, jnp.float32)
        ent_ref[...] = jnp.full((1, 1), acc_ent[0, 0] / _TOKENS, jnp.float32)


@jax.jit
def _router(hidden_states, Wr, W1, b1, W2, b2):
    x = hidden_states
    b1r = b1.reshape(1, _H4)

    out_shapes = (
        jax.ShapeDtypeStruct((_TOKENS, _E), jnp.float32),
        jax.ShapeDtypeStruct((_TOKENS, _MAXK), jnp.int32),
        jax.ShapeDtypeStruct((_TOKENS, _MAXK), jnp.float32),
        jax.ShapeDtypeStruct((1, 1), jnp.float32),
        jax.ShapeDtypeStruct((1, 1), jnp.float32),
    )
    in_specs = [
        pl.BlockSpec((_BT, _HIDDEN), lambda i: (i, 0)),
        pl.BlockSpec((_H4, _HIDDEN), lambda i: (0, 0)),
        pl.BlockSpec((1, _H4), lambda i: (0, 0)),
        pl.BlockSpec((1, _H4), lambda i: (0, 0)),
        pl.BlockSpec(memory_space=pltpu.SMEM),
        pl.BlockSpec((_E, _HIDDEN), lambda i: (0, 0)),
    ]
    out_specs = (
        pl.BlockSpec((_TOKENS, _E), lambda i: (0, 0)),
        pl.BlockSpec((_TOKENS, _MAXK), lambda i: (0, 0)),
        pl.BlockSpec((_TOKENS, _MAXK), lambda i: (0, 0)),
        pl.BlockSpec((1, 1), lambda i: (0, 0)),
        pl.BlockSpec((1, 1), lambda i: (0, 0)),
    )
    return pl.pallas_call(
        _router_body,
        grid=(_NBLK,),
        in_specs=in_specs,
        out_specs=out_specs,
        out_shape=out_shapes,
        scratch_shapes=[
            pltpu.VMEM((_WROWS, _HIDDEN), jnp.bfloat16),
            pltpu.VMEM((2, _E, _BT), jnp.float32),
            pltpu.VMEM((2, 1, _BT), jnp.float32),
            pltpu.VMEM((_E, 1), jnp.float32),
            pltpu.SMEM((1, 1), jnp.float32),
        ],
    )(x, W1, b1r, W2, b2, Wr)


def kernel(hidden_states, Wr, W1, b1, W2, b2):
    logits, idx, wts, var, ent = _router(hidden_states, Wr, W1, b1, W2, b2)
    return (logits, idx.astype(jnp.int64), wts, var[0, 0], ent[0, 0])
